# baseline (speedup 1.0000x reference)
import functools

import jax
import jax.numpy as jnp
from jax.experimental import pallas as pl
from jax.experimental.pallas import tpu as pltpu

_TOP_K = 32
_BR = 1024
_BQ = 2048


def _key_of_f32(x_f32):
    b = jax.lax.bitcast_convert_type(x_f32, jnp.int32)
    return _flip(b)


def _f32_of_key(b_i32):
    return jax.lax.bitcast_convert_type(_flip(b_i32), jnp.float32)


def _flip(b):
    return b ^ jax.lax.shift_right_arithmetic(b, 31).astype(jnp.int32) & jnp.int32(
        0x7FFFFFFF
    )


def _norm_body(s_ref, o_ref):
    x = s_ref[...]
    n = jnp.sqrt(jnp.sum(x * x, axis=1, keepdims=True))
    o_ref[...] = x / n


def _kth_threshold(sim, k):
    def step(lo, hi):
        mid = lo + jax.lax.shift_right_arithmetic(hi - lo + 1, 1)
        mid_f = _f32_of_key(mid)
        cnt = jnp.sum((sim >= mid_f).astype(jnp.int32), axis=1, keepdims=True)
        ge = cnt >= k
        lo = jnp.where(ge, mid, lo)
        hi = jnp.where(cnt == k, lo, jnp.where(ge, hi, mid - 1))
        return lo, hi

    def body(lh):
        lo, hi = lh
        for _ in range(4):
            lo, hi = step(lo, hi)
        return lo, hi

    r, n = sim.shape
    w = 128
    if n >= 2 * w and n % w == 0 and ((n // w) & (n // w - 1)) == 0 and w >= k:
        cm = sim
        while cm.shape[1] > w:
            h = cm.shape[1] // 2
            cm = jnp.maximum(cm[:, :h], cm[:, h:])
        hi = _key_of_f32(jnp.max(cm, axis=1, keepdims=True))

        def cbody(_, lh):
            lo2, hi2 = lh
            mid = lo2 + jax.lax.shift_right_arithmetic(hi2 - lo2 + 1, 1)
            cnt = jnp.sum((cm >= _f32_of_key(mid)).astype(jnp.int32),
                          axis=1, keepdims=True)
            ge = cnt >= k
            return jnp.where(ge, mid, lo2), jnp.where(ge, hi2, mid - 1)

        lo = _key_of_f32(jnp.min(cm, axis=1, keepdims=True))
        lo, _ = jax.lax.fori_loop(0, 8, cbody, (lo, hi))
    else:
        lo = _key_of_f32(jnp.min(sim, axis=1, keepdims=True))
        hi = _key_of_f32(jnp.max(sim, axis=1, keepdims=True))
    lo, _ = jax.lax.while_loop(
        lambda lh: jnp.any(lh[1] > lh[0]), body, (lo, hi))
    return _f32_of_key(lo)


def _agg_body(alpha_ref, s_ref, xs_ref, xs_blk_ref, s_blk_ref, o_ref, *, k):
    s_all = s_ref[...]
    xs_all = xs_ref[...]
    xs_blk = xs_blk_ref[...]
    s_blk = s_blk_ref[...]

    sim = jax.lax.dot_general(
        xs_blk, xs_all, (((1,), (1,)), ((), ())),
        preferred_element_type=jnp.float32,
    )
    thr = _kth_threshold(sim, k)
    mask = (sim >= thr).astype(jnp.float32)
    agg = jax.lax.dot_general(
        mask, s_all, (((1,), (0,)), ((), ())),
        preferred_element_type=jnp.float32,
    )
    o_ref[...] = s_blk + alpha_ref[0, 0] * agg


def _attn_body(alpha_ref, s_ref, q_ref, wq_ref, bq_ref, wk_ref, bk_ref,
               wv_ref, bv_ref, o_ref):
    s_all = s_ref[...]
    n = s_all.shape[0]
    half = n // 2
    pos = jnp.mean(s_all[half:, :], axis=0, keepdims=True)
    neg = jnp.mean(s_all[:half, :], axis=0, keepdims=True)

    q = q_ref[...]
    dn = (((1,), (1,)), ((), ()))
    Q = jax.lax.dot_general(q, wq_ref[...], dn,
                            preferred_element_type=jnp.float32) + bq_ref[...]
    k_pos = jax.lax.dot_general(pos, wk_ref[...], dn,
                                preferred_element_type=jnp.float32) + bk_ref[...]
    k_neg = jax.lax.dot_general(neg, wk_ref[...], dn,
                                preferred_element_type=jnp.float32) + bk_ref[...]
    v_pos = jax.lax.dot_general(pos, wv_ref[...], dn,
                                preferred_element_type=jnp.float32) + bv_ref[...]
    v_neg = jax.lax.dot_general(neg, wv_ref[...], dn,
                                preferred_element_type=jnp.float32) + bv_ref[...]

    scale = jnp.float32(q.shape[1]) ** 0.5
    l_pos = jnp.sum(Q * k_pos, axis=1, keepdims=True) / scale
    l_neg = jnp.sum(Q * k_neg, axis=1, keepdims=True) / scale
    m = jnp.maximum(l_pos, l_neg)
    e_pos = jnp.exp(l_pos - m)
    e_neg = jnp.exp(l_neg - m)
    denom = e_pos + e_neg
    ctx = (e_pos / denom) * v_pos + (e_neg / denom) * v_neg
    o_ref[...] = q + alpha_ref[0, 0] * ctx


def _run_agg(am, s_full, xs_full, xs_loc, s_loc):
    n, d = s_full.shape
    n_loc = s_loc.shape[0]
    br = _BR if n_loc % _BR == 0 else n_loc

    full = lambda r, c: pl.BlockSpec((r, c), lambda i: (0, 0))
    es = pl.pallas_call(
        functools.partial(_agg_body, k=_TOP_K),
        grid=(n_loc // br,),
        in_specs=[
            full(1, 1),
            full(n, d),
            full(n, d),
            pl.BlockSpec((br, d), lambda i: (i, 0)),
            pl.BlockSpec((br, d), lambda i: (i, 0)),
        ],
        out_specs=pl.BlockSpec((br, d), lambda i: (i, 0)),
        out_shape=jax.ShapeDtypeStruct((n_loc, d), jnp.float32),
        compiler_params=pltpu.CompilerParams(
            dimension_semantics=("arbitrary",),
        ),
    )(am, s_full, xs_full, xs_loc, s_loc)
    return es


def _run_attn(ap, s_full, q_loc, Wq, bq2, Wk, bk2, Wv, bv2):
    n, d = s_full.shape
    nq_loc = q_loc.shape[0]
    bq_blk = _BQ if nq_loc % _BQ == 0 else nq_loc
    full = lambda r, c: pl.BlockSpec((r, c), lambda i: (0, 0))
    eq = pl.pallas_call(
        _attn_body,
        grid=(nq_loc // bq_blk,),
        in_specs=[
            full(1, 1),
            full(n, d),
            pl.BlockSpec((bq_blk, d), lambda i: (i, 0)),
            full(d, d),
            full(1, d),
            full(d, d),
            full(1, d),
            full(d, d),
            full(1, d),
        ],
        out_specs=pl.BlockSpec((bq_blk, d), lambda i: (i, 0)),
        out_shape=jax.ShapeDtypeStruct((nq_loc, d), jnp.float32),
        compiler_params=pltpu.CompilerParams(
            dimension_semantics=("arbitrary",),
        ),
    )(ap, s_full, q_loc, Wq, bq2, Wk, bk2, Wv, bv2)
    return eq


def _normalize(s_emb):
    n, d = s_emb.shape
    return pl.pallas_call(
        _norm_body,
        out_shape=jax.ShapeDtypeStruct((n, d), jnp.float32),
    )(s_emb)


def kernel(s_emb, q_emb, alpha_msg, alpha_proto, Wq, bq, Wk, bk, Wv, bv):
    n, d = s_emb.shape
    nq = q_emb.shape[0]
    am = jnp.reshape(alpha_msg, (1, 1)).astype(jnp.float32)
    ap = jnp.reshape(alpha_proto, (1, 1)).astype(jnp.float32)
    bq2 = jnp.reshape(bq, (1, d))
    bk2 = jnp.reshape(bk, (1, d))
    bv2 = jnp.reshape(bv, (1, d))

    xs = _normalize(s_emb)
    es = _run_agg(am, s_emb, xs, xs, s_emb)
    eq = _run_attn(ap, s_emb, q_emb, Wq, bq2, Wk, bk2, Wv, bv2)
    return es, eq


# speedup vs baseline: 1.0295x; 1.0295x over previous
import functools

import jax
import jax.numpy as jnp
from jax.experimental import pallas as pl
from jax.experimental.pallas import tpu as pltpu

_TOP_K = 32
_BR = 512
_BQ = 2048


def _key_of_f32(x_f32):
    b = jax.lax.bitcast_convert_type(x_f32, jnp.int32)
    return _flip(b)


def _f32_of_key(b_i32):
    return jax.lax.bitcast_convert_type(_flip(b_i32), jnp.float32)


def _flip(b):
    return b ^ jax.lax.shift_right_arithmetic(b, 31).astype(jnp.int32) & jnp.int32(
        0x7FFFFFFF
    )


def _norm_body(s_ref, o_ref):
    x = s_ref[...]
    n = jnp.sqrt(jnp.sum(x * x, axis=1, keepdims=True))
    o_ref[...] = x / n


def _kth_threshold(sim, k):
    def step(lo, hi):
        mid = lo + jax.lax.shift_right_arithmetic(hi - lo + 1, 1)
        mid_f = _f32_of_key(mid)
        cnt = jnp.sum((sim >= mid_f).astype(jnp.int32), axis=1, keepdims=True)
        ge = cnt >= k
        lo = jnp.where(ge, mid, lo)
        hi = jnp.where(cnt == k, lo, jnp.where(ge, hi, mid - 1))
        return lo, hi

    def body(lh):
        lo, hi = lh
        for _ in range(4):
            lo, hi = step(lo, hi)
        return lo, hi

    r, n = sim.shape
    w = 128
    if n >= 2 * w and n % w == 0 and ((n // w) & (n // w - 1)) == 0 and w >= k:
        cm = sim
        while cm.shape[1] > w:
            h = cm.shape[1] // 2
            cm = jnp.maximum(cm[:, :h], cm[:, h:])
        hi = _key_of_f32(jnp.max(cm, axis=1, keepdims=True))

        def cbody(_, lh):
            lo2, hi2 = lh
            mid = lo2 + jax.lax.shift_right_arithmetic(hi2 - lo2 + 1, 1)
            cnt = jnp.sum((cm >= _f32_of_key(mid)).astype(jnp.int32),
                          axis=1, keepdims=True)
            ge = cnt >= k
            return jnp.where(ge, mid, lo2), jnp.where(ge, hi2, mid - 1)

        lo = _key_of_f32(jnp.min(cm, axis=1, keepdims=True))
        lo, _ = jax.lax.fori_loop(0, 8, cbody, (lo, hi))
    else:
        lo = _key_of_f32(jnp.min(sim, axis=1, keepdims=True))
        hi = _key_of_f32(jnp.max(sim, axis=1, keepdims=True))
    lo, _ = jax.lax.while_loop(
        lambda lh: jnp.any(lh[1] > lh[0]), body, (lo, hi))
    return _f32_of_key(lo)


def _agg_body(alpha_ref, s_ref, xs_ref, xs_blk_ref, s_blk_ref, o_ref, *, k):
    s_all = s_ref[...]
    xs_all = xs_ref[...]
    xs_blk = xs_blk_ref[...]
    s_blk = s_blk_ref[...]

    sim = jax.lax.dot_general(
        xs_blk, xs_all, (((1,), (1,)), ((), ())),
        preferred_element_type=jnp.float32,
    )
    thr = _kth_threshold(sim, k)
    mask = (sim >= thr).astype(jnp.float32)
    agg = jax.lax.dot_general(
        mask, s_all, (((1,), (0,)), ((), ())),
        preferred_element_type=jnp.float32,
    )
    o_ref[...] = s_blk + alpha_ref[0, 0] * agg


def _attn_body(alpha_ref, s_ref, q_ref, wq_ref, bq_ref, wk_ref, bk_ref,
               wv_ref, bv_ref, o_ref):
    s_all = s_ref[...]
    n = s_all.shape[0]
    half = n // 2
    pos = jnp.mean(s_all[half:, :], axis=0, keepdims=True)
    neg = jnp.mean(s_all[:half, :], axis=0, keepdims=True)

    q = q_ref[...]
    dn = (((1,), (1,)), ((), ()))
    Q = jax.lax.dot_general(q, wq_ref[...], dn,
                            preferred_element_type=jnp.float32) + bq_ref[...]
    k_pos = jax.lax.dot_general(pos, wk_ref[...], dn,
                                preferred_element_type=jnp.float32) + bk_ref[...]
    k_neg = jax.lax.dot_general(neg, wk_ref[...], dn,
                                preferred_element_type=jnp.float32) + bk_ref[...]
    v_pos = jax.lax.dot_general(pos, wv_ref[...], dn,
                                preferred_element_type=jnp.float32) + bv_ref[...]
    v_neg = jax.lax.dot_general(neg, wv_ref[...], dn,
                                preferred_element_type=jnp.float32) + bv_ref[...]

    scale = jnp.float32(q.shape[1]) ** 0.5
    l_pos = jnp.sum(Q * k_pos, axis=1, keepdims=True) / scale
    l_neg = jnp.sum(Q * k_neg, axis=1, keepdims=True) / scale
    m = jnp.maximum(l_pos, l_neg)
    e_pos = jnp.exp(l_pos - m)
    e_neg = jnp.exp(l_neg - m)
    denom = e_pos + e_neg
    ctx = (e_pos / denom) * v_pos + (e_neg / denom) * v_neg
    o_ref[...] = q + alpha_ref[0, 0] * ctx


def _run_agg(am, s_full, xs_full, xs_loc, s_loc):
    n, d = s_full.shape
    n_loc = s_loc.shape[0]
    br = _BR if n_loc % _BR == 0 else n_loc

    full = lambda r, c: pl.BlockSpec((r, c), lambda i: (0, 0))
    es = pl.pallas_call(
        functools.partial(_agg_body, k=_TOP_K),
        grid=(n_loc // br,),
        in_specs=[
            full(1, 1),
            full(n, d),
            full(n, d),
            pl.BlockSpec((br, d), lambda i: (i, 0)),
            pl.BlockSpec((br, d), lambda i: (i, 0)),
        ],
        out_specs=pl.BlockSpec((br, d), lambda i: (i, 0)),
        out_shape=jax.ShapeDtypeStruct((n_loc, d), jnp.float32),
        compiler_params=pltpu.CompilerParams(
            dimension_semantics=("arbitrary",),
        ),
    )(am, s_full, xs_full, xs_loc, s_loc)
    return es


def _run_attn(ap, s_full, q_loc, Wq, bq2, Wk, bk2, Wv, bv2):
    n, d = s_full.shape
    nq_loc = q_loc.shape[0]
    bq_blk = _BQ if nq_loc % _BQ == 0 else nq_loc
    full = lambda r, c: pl.BlockSpec((r, c), lambda i: (0, 0))
    eq = pl.pallas_call(
        _attn_body,
        grid=(nq_loc // bq_blk,),
        in_specs=[
            full(1, 1),
            full(n, d),
            pl.BlockSpec((bq_blk, d), lambda i: (i, 0)),
            full(d, d),
            full(1, d),
            full(d, d),
            full(1, d),
            full(d, d),
            full(1, d),
        ],
        out_specs=pl.BlockSpec((bq_blk, d), lambda i: (i, 0)),
        out_shape=jax.ShapeDtypeStruct((nq_loc, d), jnp.float32),
        compiler_params=pltpu.CompilerParams(
            dimension_semantics=("arbitrary",),
        ),
    )(ap, s_full, q_loc, Wq, bq2, Wk, bk2, Wv, bv2)
    return eq


def _normalize(s_emb):
    n, d = s_emb.shape
    return pl.pallas_call(
        _norm_body,
        out_shape=jax.ShapeDtypeStruct((n, d), jnp.float32),
    )(s_emb)


def kernel(s_emb, q_emb, alpha_msg, alpha_proto, Wq, bq, Wk, bk, Wv, bv):
    n, d = s_emb.shape
    nq = q_emb.shape[0]
    am = jnp.reshape(alpha_msg, (1, 1)).astype(jnp.float32)
    ap = jnp.reshape(alpha_proto, (1, 1)).astype(jnp.float32)
    bq2 = jnp.reshape(bq, (1, d))
    bk2 = jnp.reshape(bk, (1, d))
    bv2 = jnp.reshape(bv, (1, d))

    xs = _normalize(s_emb)
    es = _run_agg(am, s_emb, xs, xs, s_emb)
    eq = _run_attn(ap, s_emb, q_emb, Wq, bq2, Wk, bk2, Wv, bv2)
    return es, eq
